# baseline (speedup 1.0000x reference)
import functools

import jax
import jax.numpy as jnp
from jax import lax
from jax.experimental import pallas as pl
from jax.experimental.pallas import tpu as pltpu
from jax.experimental.pallas import tpu_sc as plsc

ALPHA = 0.5
ITERS = 10
NC = 2
NS = 16
W = 128
SW = 2 * W
LANES = 16
NBUF = 4
CH = 8


def _sc_step_body(n_sw, rows_per_sub, d_reg,
                  out_hbm, col_hbm, row_hbm, wa_hbm, parts_hbm,
                  acc, cc0r, cc1r, rc0, rc1, wc0, wc1,
                  g0, g1, g2, g3,
                  gs0, gs1, gs2, gs3, ss0, ss1, ss2, ss3, cs0, cs1):
  gbufs = (g0, g1, g2, g3)
  gsems = (gs0, gs1, gs2, gs3)
  ssems = (ss0, ss1, ss2, ss3)
  colcs = (cc0r, cc1r)
  rcs = (rc0, rc1)
  wcs = (wc0, wc1)
  csems = (cs0, cs1)
  c = lax.axis_index("c")
  s = lax.axis_index("s")
  widx = c * NS + s
  n_chunk = n_sw // CH
  dh = d_reg * LANES

  def ghalf(t, h):
    return gbufs[t].at[pl.ds(h * W, W)]

  def gather_start(t, tc, r):
    for h in range(2):
      pltpu.async_copy(out_hbm.at[colcs[tc].at[r + h]], ghalf(t, h),
                       gsems[t])

  def gather_wait(t):
    for h in range(2):
      pltpu.make_async_copy(out_hbm.at[rc0.at[0]], ghalf(t, h),
                            gsems[t]).wait()

  def scatter_start(t, tc, r):
    for h in range(2):
      pltpu.async_copy(ghalf(t, h), acc.at[rcs[tc].at[r + h]], ssems[t],
                       add=True)

  def scatter_wait(t):
    for h in range(2):
      pltpu.make_async_copy(ghalf(t, h), acc.at[rc0.at[0]],
                            ssems[t]).wait()

  def chunk_start(cc, tc):
    rows = pl.ds(cc * 2 * CH, 2 * CH)
    pltpu.async_copy(col_hbm.at[widx].at[rows], colcs[tc], csems[tc])
    pltpu.async_copy(row_hbm.at[widx].at[rows], rcs[tc], csems[tc])
    pltpu.async_copy(wa_hbm.at[widx].at[rows], wcs[tc], csems[tc])

  def chunk_wait(cc, tc):
    rows = pl.ds(cc * 2 * CH, 2 * CH)
    pltpu.make_async_copy(col_hbm.at[widx].at[rows], colcs[tc],
                          csems[tc]).wait()
    pltpu.make_async_copy(row_hbm.at[widx].at[rows], rcs[tc],
                          csems[tc]).wait()
    pltpu.make_async_copy(wa_hbm.at[widx].at[rows], wcs[tc],
                          csems[tc]).wait()

  chunk_start(0, 0)
  chunk_wait(0, 0)
  gather_start(0, 0, 0)
  gather_start(1, 0, 2)

  zv = jnp.zeros((LANES,), jnp.float32)

  @pl.loop(0, SW)
  def _zrow(r):
    for k in range(d_reg):
      g3[r, pl.ds(k * LANES, LANES)] = zv

  base = s * rows_per_sub
  nfull = rows_per_sub // SW
  for z in range(nfull):
    pltpu.sync_copy(g3, acc.at[pl.ds(base + z * SW, SW)])
  rem = rows_per_sub - nfull * SW
  if rem:
    pltpu.sync_copy(g3.at[pl.ds(0, rem)],
                    acc.at[pl.ds(base + nfull * SW, rem)])
  plsc.subcore_barrier()

  nblk = SW // LANES

  def scale_sw(jcs, t, tc):
    @pl.loop(0, nblk)
    def _blk(b):
      wv = wcs[tc][2 * jcs + b // (W // LANES),
                   pl.ds((b % (W // LANES)) * LANES, LANES)]
      for e16 in range(LANES):
        ws = wv.at[jnp.full((LANES,), e16, jnp.int32)].get(
            mode="promise_in_bounds")
        e = b * LANES + e16
        for k in range(d_reg):
          sl = pl.ds(k * LANES, LANES)
          gbufs[t][e, sl] = gbufs[t][e, sl] * ws

  @pl.loop(0, n_chunk, step=2)
  def _chunkpair(cc0):
    for ccs in range(2):
      cc = cc0 + ccs
      tcur = ccs
      for jcs in range(CH):
        j = cc * CH + jcs
        t = jcs % NBUF
        t2 = (t + 2) % NBUF
        gather_wait(t)

        @pl.when(j >= 2)
        def _():
          scatter_wait(t2)

        @pl.when(j + 2 < n_sw)
        def _():
          jp = jcs + 2
          if jp < CH:
            gather_start(t2, tcur, 2 * jp)
          else:
            gather_start(t2, 1 - tcur, 2 * (jp - CH))

        if jcs == 2:
          @pl.when(cc + 1 < n_chunk)
          def _():
            chunk_start(cc + 1, 1 - tcur)

        scale_sw(jcs, t, tcur)

        if jcs == 5:
          @pl.when(cc + 1 < n_chunk)
          def _():
            chunk_wait(cc + 1, 1 - tcur)

        scatter_start(t, tcur, 2 * jcs)

  scatter_wait((n_sw - 2) % NBUF)
  scatter_wait((n_sw - 1) % NBUF)

  plsc.subcore_barrier()

  for z in range(nfull):
    zr = pl.ds(base + z * SW, SW)
    pltpu.sync_copy(acc.at[zr], gbufs[z % 2])
    pltpu.sync_copy(gbufs[z % 2], parts_hbm.at[c].at[zr])
  if rem:
    zr = pl.ds(base + nfull * SW, rem)
    pltpu.sync_copy(acc.at[zr], g2.at[pl.ds(0, rem)])
    pltpu.sync_copy(g2.at[pl.ds(0, rem)], parts_hbm.at[c].at[zr])


def _prep_body(x_ref, w_ref, res_ref, wa_ref):
  res_ref[...] = x_ref[...] * (1.0 - ALPHA)
  wa_ref[...] = w_ref[...] * ALPHA


def _combine_body(n_nodes, dh, p_ref, res_ref, o_ref):
  o_ref[0] = p_ref[0, :n_nodes] + res_ref[:, :dh]
  o_ref[1] = p_ref[1, :n_nodes] + res_ref[:, dh:]


def kernel(x, edge_index, edge_weight):
  n_nodes, d = x.shape
  n_edges = edge_weight.shape[0]
  dh = d // NC
  d_reg = dh // LANES

  epw = -(-n_edges // NS)
  n_sw = -(-epw // SW)
  n_sw = -(-n_sw // (2 * CH)) * (2 * CH)
  e_pad = NS * n_sw * SW
  pad = e_pad - n_edges
  rows_per_sub = -(-(-(-n_nodes // NS)) // 8) * 8
  n_rows_pad = NS * rows_per_sub
  nrows = n_sw * 2

  spread = (jnp.arange(pad, dtype=jnp.int32) * 97) % n_nodes
  colp = jnp.concatenate([edge_index[1], spread])
  rowp = jnp.concatenate([edge_index[0], spread])
  wp = jnp.concatenate([edge_weight, jnp.zeros((pad,), edge_weight.dtype)])
  col_a = jnp.concatenate([colp, colp + n_nodes]).reshape(NC * NS, nrows, W)
  row_a = jnp.concatenate([rowp, rowp]).reshape(NC * NS, nrows, W)
  w_flat = jnp.concatenate([wp, wp]).reshape(NC * NS, nrows, W)

  res, wa_a = pl.pallas_call(
      _prep_body,
      out_shape=(
          jax.ShapeDtypeStruct((n_nodes, d), jnp.float32),
          jax.ShapeDtypeStruct((NC * NS, nrows, W), jnp.float32),
      ),
  )(x, w_flat)

  mesh = plsc.VectorSubcoreMesh(core_axis_name="c", subcore_axis_name="s")
  sc_step = pl.kernel(
      functools.partial(_sc_step_body, n_sw, rows_per_sub, d_reg),
      out_type=jax.ShapeDtypeStruct((NC, n_rows_pad, dh), jnp.float32),
      mesh=mesh,
      compiler_params=pltpu.CompilerParams(use_tc_tiling_on_sc=False),
      scratch_types=[
          pltpu.VMEM_SHARED((n_rows_pad, dh), jnp.float32),
          pltpu.VMEM((2 * CH, W), jnp.int32),
          pltpu.VMEM((2 * CH, W), jnp.int32),
          pltpu.VMEM((2 * CH, W), jnp.int32),
          pltpu.VMEM((2 * CH, W), jnp.int32),
          pltpu.VMEM((2 * CH, W), jnp.float32),
          pltpu.VMEM((2 * CH, W), jnp.float32),
          pltpu.VMEM((SW, dh), jnp.float32),
          pltpu.VMEM((SW, dh), jnp.float32),
          pltpu.VMEM((SW, dh), jnp.float32),
          pltpu.VMEM((SW, dh), jnp.float32),
          pltpu.SemaphoreType.DMA,
          pltpu.SemaphoreType.DMA,
          pltpu.SemaphoreType.DMA,
          pltpu.SemaphoreType.DMA,
          pltpu.SemaphoreType.DMA,
          pltpu.SemaphoreType.DMA,
          pltpu.SemaphoreType.DMA,
          pltpu.SemaphoreType.DMA,
          pltpu.SemaphoreType.DMA,
          pltpu.SemaphoreType.DMA,
      ],
  )

  combine = pl.pallas_call(
      functools.partial(_combine_body, n_nodes, dh),
      out_shape=jax.ShapeDtypeStruct((NC, n_nodes, dh), jnp.float32),
  )

  out2 = jnp.concatenate([x[:, :dh], x[:, dh:]], axis=0)
  for _ in range(ITERS):
    parts = sc_step(out2, col_a, row_a, wa_a)
    out2 = combine(parts, res).reshape(NC * n_nodes, dh)
  out2 = out2.reshape(NC, n_nodes, dh)
  return jnp.concatenate([out2[0], out2[1]], axis=1)


# speedup vs baseline: 2.5783x; 2.5783x over previous
import functools

import jax
import jax.numpy as jnp
from jax import lax
from jax.experimental import pallas as pl
from jax.experimental.pallas import tpu as pltpu
from jax.experimental.pallas import tpu_sc as plsc

ALPHA = 0.5
ITERS = 10
NC = 2
NS = 16
NW = NC * NS
W = 128
LANES = 16
NBUF = 2
CH = 8


def _sc_step_body(n_win, rows_per_sub, d_reg,
                  out_hbm, col_hbm, row_hbm, wa_hbm, parts_hbm,
                  acc, colv, rc0, rc1, wc0, wc1,
                  g0, g1, gs0, gs1, ss0, ss1, rs0, rs1):
  gbufs = (g0, g1)
  gsems = (gs0, gs1)
  ssems = (ss0, ss1)
  rcs = (rc0, rc1)
  wcs = (wc0, wc1)
  rsems = (rs0, rs1)
  c = lax.axis_index("c")
  s = lax.axis_index("s")
  widx = c * NS + s
  n_chunk = n_win // CH

  def gather_start(j, t):
    pltpu.async_copy(out_hbm.at[colv.at[j]], gbufs[t], gsems[t])

  def gather_wait(j, t):
    pltpu.make_async_copy(out_hbm.at[colv.at[j]], gbufs[t], gsems[t]).wait()

  def scatter_start(j, jc, t, tc):
    pltpu.async_copy(gbufs[t], acc.at[rcs[tc].at[jc]], ssems[t], add=True)

  def scatter_wait(j, jc, t, tc):
    pltpu.make_async_copy(
        gbufs[t], acc.at[rcs[tc].at[jc]], ssems[t]).wait()

  def chunk_start(cc, tc):
    pltpu.async_copy(row_hbm.at[widx].at[pl.ds(cc * CH, CH)], rcs[tc],
                     rsems[tc])
    pltpu.async_copy(wa_hbm.at[widx].at[pl.ds(cc * CH, CH)], wcs[tc],
                     rsems[tc])

  def chunk_wait(cc, tc):
    pltpu.make_async_copy(row_hbm.at[widx].at[pl.ds(cc * CH, CH)], rcs[tc],
                          rsems[tc]).wait()
    pltpu.make_async_copy(wa_hbm.at[widx].at[pl.ds(cc * CH, CH)], wcs[tc],
                          rsems[tc]).wait()

  pltpu.sync_copy(col_hbm.at[widx], colv)
  gather_start(0, 0)
  chunk_start(0, 0)

  zv = jnp.zeros((LANES,), jnp.float32)

  @pl.loop(0, W)
  def _zrow(r):
    for k in range(d_reg):
      g1[r, pl.ds(k * LANES, LANES)] = zv

  base = s * rows_per_sub
  nfull = rows_per_sub // W
  for z in range(nfull):
    pltpu.sync_copy(g1, acc.at[pl.ds(base + z * W, W)])
  rem = rows_per_sub - nfull * W
  if rem:
    pltpu.sync_copy(g1.at[pl.ds(0, rem)],
                    acc.at[pl.ds(base + nfull * W, rem)])
  chunk_wait(0, 0)
  plsc.subcore_barrier()

  nblk = W // LANES

  def scale_blocks(jc, t, tc, b_lo, b_hi):
    @pl.loop(b_lo, b_hi)
    def _blk(b):
      wv = wcs[tc][jc, pl.ds(b * LANES, LANES)]
      for e16 in range(LANES):
        ws = wv.at[jnp.full((LANES,), e16, jnp.int32)].get(
            mode="promise_in_bounds")
        e = b * LANES + e16
        for k in range(d_reg):
          sl = pl.ds(k * LANES, LANES)
          gbufs[t][e, sl] = gbufs[t][e, sl] * ws

  @pl.loop(0, n_chunk, step=2)
  def _chunkpair(cc0):
    for ccs in range(2):
      cc = cc0 + ccs
      tcur = ccs
      for jcs in range(CH):
        j = cc * CH + jcs
        t = jcs % NBUF
        to = 1 - t
        gather_wait(j, t)

        @pl.when(j >= 1)
        def _():
          if jcs == 0:
            scatter_wait(j - 1, CH - 1, to, 1 - tcur)
          else:
            scatter_wait(j - 1, jcs - 1, to, tcur)

        @pl.when(j + 1 < n_win)
        def _():
          gather_start(j + 1, to)

        if jcs == 1:
          @pl.when(cc + 1 < n_chunk)
          def _():
            chunk_start(cc + 1, 1 - tcur)

        scale_blocks(jcs, t, tcur, 0, nblk)
        scatter_start(j, jcs, t, tcur)

      @pl.when(cc + 1 < n_chunk)
      def _():
        chunk_wait(cc + 1, 1 - tcur)

  scatter_wait(n_win - 1, CH - 1, (n_win - 1) % NBUF, (n_chunk - 1) % 2)

  plsc.subcore_barrier()

  for z in range(nfull):
    zr = pl.ds(base + z * W, W)
    pltpu.sync_copy(acc.at[zr], gbufs[z % 2])
    pltpu.sync_copy(gbufs[z % 2], parts_hbm.at[c].at[zr])
  if rem:
    zr = pl.ds(base + nfull * W, rem)
    pltpu.sync_copy(acc.at[zr], g1.at[pl.ds(0, rem)])
    pltpu.sync_copy(g1.at[pl.ds(0, rem)], parts_hbm.at[c].at[zr])


def _prep_body(x_ref, w_ref, res_ref, wa_ref):
  res_ref[...] = x_ref[...] * (1.0 - ALPHA)
  wa_ref[...] = w_ref[...] * ALPHA


def _combine_body(n_nodes, p_ref, res_ref, o_ref):
  o_ref[...] = p_ref[0, :n_nodes] + p_ref[1, :n_nodes] + res_ref[...]


def kernel(x, edge_index, edge_weight):
  n_nodes, d = x.shape
  n_edges = edge_weight.shape[0]
  d_reg = d // LANES

  epw = -(-n_edges // NW)
  n_win = -(-epw // W)
  n_win = -(-n_win // (2 * CH)) * (2 * CH)
  e_pad = NW * n_win * W
  pad = e_pad - n_edges
  rows_per_sub = -(-(-(-n_nodes // NS)) // 8) * 8
  n_rows_pad = NS * rows_per_sub

  spread = (jnp.arange(pad, dtype=jnp.int32) * 97) % n_nodes
  col_a = jnp.concatenate([edge_index[1], spread]).reshape(NW, n_win, W)
  row_a = jnp.concatenate([edge_index[0], spread]).reshape(NW, n_win, W)
  w_flat = jnp.concatenate(
      [edge_weight, jnp.zeros((pad,), edge_weight.dtype)]
  ).reshape(NW, n_win, W)

  res, wa_a = pl.pallas_call(
      _prep_body,
      out_shape=(
          jax.ShapeDtypeStruct((n_nodes, d), jnp.float32),
          jax.ShapeDtypeStruct((NW, n_win, W), jnp.float32),
      ),
  )(x, w_flat)

  mesh = plsc.VectorSubcoreMesh(core_axis_name="c", subcore_axis_name="s")
  sc_step = pl.kernel(
      functools.partial(_sc_step_body, n_win, rows_per_sub, d_reg),
      out_type=jax.ShapeDtypeStruct((NC, n_rows_pad, d), jnp.float32),
      mesh=mesh,
      scratch_types=[
          pltpu.VMEM_SHARED((n_rows_pad, d), jnp.float32),
          pltpu.VMEM((n_win, W), jnp.int32),
          pltpu.VMEM((CH, W), jnp.int32),
          pltpu.VMEM((CH, W), jnp.int32),
          pltpu.VMEM((CH, W), jnp.float32),
          pltpu.VMEM((CH, W), jnp.float32),
          pltpu.VMEM((W, d), jnp.float32),
          pltpu.VMEM((W, d), jnp.float32),
          pltpu.SemaphoreType.DMA,
          pltpu.SemaphoreType.DMA,
          pltpu.SemaphoreType.DMA,
          pltpu.SemaphoreType.DMA,
          pltpu.SemaphoreType.DMA,
          pltpu.SemaphoreType.DMA,
      ],
  )

  combine = pl.pallas_call(
      functools.partial(_combine_body, n_nodes),
      out_shape=jax.ShapeDtypeStruct((n_nodes, d), jnp.float32),
  )

  out = x
  for _ in range(ITERS):
    parts = sc_step(out, col_a, row_a, wa_a)
    out = combine(parts, res)
  return out
